# Initial kernel scaffold; baseline (speedup 1.0000x reference)
import jax
import jax.numpy as jnp
from jax.experimental import pallas as pl


def kernel(points, rotations):
    raise NotImplementedError("write your pallas kernel here")


# speedup vs baseline: 7.2027x; 7.2027x over previous
import functools

import jax
import jax.numpy as jnp
from jax import lax
from jax.experimental import pallas as pl
from jax.experimental.pallas import tpu as pltpu
from jax.experimental.pallas import tpu_sc as plsc

N = 5000
D = 32
B = 2
NBINS = 10
BIN = 500
SBIN = 512
K = 5
P = NBINS * SBIN
PAD = P - N
GPB = SBIN - BIN
RB = 200
NW = 32
RPW = (B * P) // NW
CH = 4
CW = RPW // CH


def _k1_body(pts_ref, mul_ref, pos_ref, comb_ref):
    b = pl.program_id(0)
    pts = pts_ref[0]
    mul = mul_ref[0]
    cmul = jnp.concatenate([mul, -mul], axis=1)
    m = jnp.max(cmul, axis=1, keepdims=True)
    li10 = lax.broadcasted_iota(jnp.int32, (N, NBINS), 1)
    binr = jnp.min(jnp.where(cmul == m, li10, 100), axis=1, keepdims=True)
    li16 = lax.broadcasted_iota(jnp.int32, (N, 16), 1)
    oh = jnp.where(li16 == binr, 1.0, 0.0)
    cs = oh
    s = 1
    while s < N:
        shifted = jnp.concatenate(
            [jnp.zeros((s, 16), jnp.float32), cs[: N - s, :]], axis=0)
        cs = cs + shifted
        s *= 2
    excl = cs - oh
    counts = jnp.sum(oh, axis=0, keepdims=True)
    st = counts
    s = 1
    while s < 16:
        shifted = jnp.concatenate(
            [jnp.zeros((1, s), jnp.float32), st[:, : 16 - s]], axis=1)
        st = st + shifted
        s *= 2
    starts = st - counts
    pos = jnp.sum((excl + starts) * oh, axis=1, keepdims=True)
    pi = pos.astype(jnp.int32)
    pos_ref[0] = SBIN * (pi // BIN) + pi % BIN + P * b

    idcol = lax.broadcasted_iota(jnp.int32, (P, 1), 0).astype(jnp.float32)
    pts_pad = jnp.concatenate(
        [pts, jnp.zeros((PAD, D), jnp.float32)], axis=0)
    comb_ref[...] = jnp.concatenate(
        [pts_pad, idcol, jnp.zeros((P, 128 - D - 1), jnp.float32)], axis=1)


def _k1_call(points, mul):
    return pl.pallas_call(
        _k1_body,
        grid=(B,),
        in_specs=[
            pl.BlockSpec((1, N, D), lambda b: (b, 0, 0)),
            pl.BlockSpec((1, N, 5), lambda b: (b, 0, 0)),
        ],
        out_specs=[
            pl.BlockSpec((1, N, 1), lambda b: (b, 0, 0)),
            pl.BlockSpec((P, 128), lambda b: (b, 0)),
        ],
        out_shape=[
            jax.ShapeDtypeStruct((B, N, 1), jnp.int32),
            jax.ShapeDtypeStruct((B * P, 128), jnp.float32),
        ],
    )(points, mul)


def _sc_scatter(idx, data):
    mesh = plsc.VectorSubcoreMesh(core_axis_name="c", subcore_axis_name="s")

    @functools.partial(
        pl.kernel,
        mesh=mesh,
        out_type=jax.ShapeDtypeStruct((B * P, 128), jnp.float32),
        scratch_types=[
            pltpu.VMEM((CH, CW), jnp.int32),
            pltpu.VMEM((RPW, 128), jnp.float32),
            pltpu.SemaphoreType.DMA,
        ],
    )
    def scat(idx_hbm, data_hbm, out_hbm, idx_v, rows_v, sem):
        wid = lax.axis_index("s") * 2 + lax.axis_index("c")
        base = wid * RPW
        for j in range(CH):
            pltpu.sync_copy(idx_hbm.at[pl.ds(base + j * CW, CW)], idx_v.at[j])
        pltpu.sync_copy(data_hbm.at[pl.ds(base, RPW)], rows_v)
        copies = []
        for j in range(CH):
            copies.append(
                pltpu.async_copy(
                    rows_v.at[pl.ds(j * CW, CW)], out_hbm.at[idx_v.at[j]], sem))
        for c in copies:
            c.wait()

    return scat(idx, data)


def _k3_body(sc_ref, erow_ref, eidx_ref):
    b = pl.program_id(0)
    i = pl.program_id(1)
    blk = sc_ref[0]
    pts = blk[:, :D]
    ids = blk[:, D:D + 1]
    g = lax.dot_general(pts, pts, (((1,), (1,)), ((), ())),
                        preferred_element_type=jnp.float32)
    na = jnp.sum(pts * pts, axis=1, keepdims=True)
    ii = jnp.where(
        lax.broadcasted_iota(jnp.int32, (SBIN, SBIN), 0)
        == lax.broadcasted_iota(jnp.int32, (SBIN, SBIN), 1), 1.0, 0.0)
    nb = jnp.sum(ii * na, axis=0, keepdims=True)
    idsr = jnp.sum(ii * ids, axis=0, keepdims=True)
    iota_l = lax.broadcasted_iota(jnp.int32, (SBIN, SBIN), 1)
    d2 = jnp.maximum(na - 2.0 * g + nb, 1e-6)
    d2 = jnp.where(iota_l < BIN, d2, 3.0e38)
    idsb = jnp.broadcast_to(idsr, (SBIN, SBIN))
    work = d2
    vals = []
    cols = []
    for _ in range(K):
        mn = jnp.min(work, axis=1, keepdims=True)
        loc = jnp.min(jnp.where(work == mn, iota_l, 10 ** 9), axis=1,
                      keepdims=True)
        sel = iota_l == loc
        colg = jnp.sum(jnp.where(sel, idsb, 0.0), axis=1, keepdims=True)
        vals.append(mn)
        cols.append(colg)
        work = jnp.where(sel, 3.4e38, work)
    d2k = jnp.concatenate(vals, axis=1)
    dmv = jnp.exp(-0.1 * jnp.sqrt(d2k))
    colk = jnp.concatenate(cols, axis=1)
    erow_ref[0] = jnp.concatenate(
        [dmv, colk, jnp.zeros((SBIN, 128 - 2 * K), jnp.float32)], axis=1)
    li = lax.broadcasted_iota(jnp.int32, (1, SBIN), 1)
    eidx_ref[0, 0] = jnp.where(
        li < BIN, idsr.astype(jnp.int32), N + GPB * i + (li - BIN)) + P * b


def _k3_call(sorted_comb):
    return pl.pallas_call(
        _k3_body,
        grid=(B, NBINS),
        in_specs=[pl.BlockSpec((1, SBIN, 128), lambda b, i: (b, i, 0))],
        out_specs=[
            pl.BlockSpec((1, SBIN, 128), lambda b, i: (b, i, 0)),
            pl.BlockSpec((1, 1, 1, SBIN), lambda b, i: (b, i, 0, 0)),
        ],
        out_shape=[
            jax.ShapeDtypeStruct((B, P, 128), jnp.float32),
            jax.ShapeDtypeStruct((B, NBINS, 1, SBIN), jnp.int32),
        ],
    )(sorted_comb)


def _e_body(ge_ref, out_ref):
    ent = ge_ref[0]
    iota_l = lax.broadcasted_iota(jnp.int32, (RB, N), 1)
    acc = jnp.zeros((RB, N), jnp.float32)
    for j in range(K):
        v = ent[:, j:j + 1]
        c = ent[:, K + j:K + j + 1].astype(jnp.int32)
        acc = acc + jnp.where(iota_l == c, v, 0.0)
    out_ref[0] = acc


def _e_call(ge):
    return pl.pallas_call(
        _e_body,
        grid=(B, N // RB),
        in_specs=[pl.BlockSpec((1, RB, 128), lambda b, r: (b, r, 0))],
        out_specs=pl.BlockSpec((1, RB, N), lambda b, r: (b, r, 0)),
        out_shape=jax.ShapeDtypeStruct((B, N, N), jnp.float32),
        compiler_params=pltpu.CompilerParams(
            dimension_semantics=("parallel", "parallel")),
    )(ge)


def kernel(points, rotations):
    rot5 = rotations[:, : NBINS // 2]
    mul = jnp.matmul(points, rot5)

    pos, comb = _k1_call(points, mul)
    t = jnp.arange(PAD, dtype=jnp.int32)
    padi = (SBIN * (t // GPB) + BIN + t % GPB)[None, :] \
        + P * jnp.arange(B, dtype=jnp.int32)[:, None]
    pflat = jnp.concatenate(
        [pos.reshape(B, N), padi], axis=1).reshape(B * P)
    sorted_comb = _sc_scatter(pflat, comb)

    erows, eidx = _k3_call(sorted_comb.reshape(B, P, 128))
    ge = _sc_scatter(eidx.reshape(B * P), erows.reshape(B * P, 128))

    ge_u = ge.reshape(B, P, 128)[:, :N, :]
    return _e_call(ge_u)


# speedup vs baseline: 8.8910x; 1.2344x over previous
import functools

import jax
import jax.numpy as jnp
from jax import lax
from jax.experimental import pallas as pl
from jax.experimental.pallas import tpu as pltpu
from jax.experimental.pallas import tpu_sc as plsc

N = 5000
D = 32
B = 2
NBINS = 10
BIN = 500
SBIN = 512
K = 5
P = NBINS * SBIN
PAD = P - N
GPB = SBIN - BIN
RB = 1000
NW = 32
RPW = (B * P) // NW
CH = 4
CW = RPW // CH


def _k1_body(pts_ref, mul_ref, pos_ref, comb_ref):
    b = pl.program_id(0)
    pts = pts_ref[0]
    mul = mul_ref[0]
    cmul = jnp.concatenate([mul, -mul], axis=1)
    m = jnp.max(cmul, axis=1, keepdims=True)
    li10 = lax.broadcasted_iota(jnp.int32, (N, NBINS), 1)
    binr = jnp.min(jnp.where(cmul == m, li10, 100), axis=1, keepdims=True)
    li16 = lax.broadcasted_iota(jnp.int32, (N, 16), 1)
    oh = jnp.where(li16 == binr, 1.0, 0.0)
    cs = oh
    s = 1
    while s < N:
        shifted = jnp.concatenate(
            [jnp.zeros((s, 16), jnp.float32), cs[: N - s, :]], axis=0)
        cs = cs + shifted
        s *= 2
    excl = cs - oh
    counts = jnp.sum(oh, axis=0, keepdims=True)
    st = counts
    s = 1
    while s < 16:
        shifted = jnp.concatenate(
            [jnp.zeros((1, s), jnp.float32), st[:, : 16 - s]], axis=1)
        st = st + shifted
        s *= 2
    starts = st - counts
    pos = jnp.sum((excl + starts) * oh, axis=1, keepdims=True)
    pi = pos.astype(jnp.int32)
    pos_ref[0] = SBIN * (pi // BIN) + pi % BIN + P * b

    idcol = lax.broadcasted_iota(jnp.int32, (P, 1), 0).astype(jnp.float32)
    pts_pad = jnp.concatenate(
        [pts, jnp.zeros((PAD, D), jnp.float32)], axis=0)
    comb_ref[...] = jnp.concatenate(
        [pts_pad, idcol, jnp.zeros((P, 128 - D - 1), jnp.float32)], axis=1)


def _k1_call(points, mul):
    return pl.pallas_call(
        _k1_body,
        grid=(B,),
        in_specs=[
            pl.BlockSpec((1, N, D), lambda b: (b, 0, 0)),
            pl.BlockSpec((1, N, 5), lambda b: (b, 0, 0)),
        ],
        out_specs=[
            pl.BlockSpec((1, N, 1), lambda b: (b, 0, 0)),
            pl.BlockSpec((P, 128), lambda b: (b, 0)),
        ],
        out_shape=[
            jax.ShapeDtypeStruct((B, N, 1), jnp.int32),
            jax.ShapeDtypeStruct((B * P, 128), jnp.float32),
        ],
    )(points, mul)


def _sc_scatter(idx, data):
    mesh = plsc.VectorSubcoreMesh(core_axis_name="c", subcore_axis_name="s")

    @functools.partial(
        pl.kernel,
        mesh=mesh,
        out_type=jax.ShapeDtypeStruct((B * P, 128), jnp.float32),
        scratch_types=[
            pltpu.VMEM((CH, CW), jnp.int32),
            pltpu.VMEM((RPW, 128), jnp.float32),
            pltpu.SemaphoreType.DMA,
        ],
    )
    def scat(idx_hbm, data_hbm, out_hbm, idx_v, rows_v, sem):
        wid = lax.axis_index("s") * 2 + lax.axis_index("c")
        base = wid * RPW
        for j in range(CH):
            pltpu.sync_copy(idx_hbm.at[pl.ds(base + j * CW, CW)], idx_v.at[j])
        pltpu.sync_copy(data_hbm.at[pl.ds(base, RPW)], rows_v)
        copies = []
        for j in range(CH):
            copies.append(
                pltpu.async_copy(
                    rows_v.at[pl.ds(j * CW, CW)], out_hbm.at[idx_v.at[j]], sem))
        for c in copies:
            c.wait()

    return scat(idx, data)


def _k3_body(sc_ref, erow_ref, eidx_ref):
    b = pl.program_id(0)
    i = pl.program_id(1)
    blk = sc_ref[0]
    pts = blk[:, :D]
    ids = blk[:, D:D + 1]
    g = lax.dot_general(pts, pts, (((1,), (1,)), ((), ())),
                        preferred_element_type=jnp.float32)
    na = jnp.sum(pts * pts, axis=1, keepdims=True)
    ii = jnp.where(
        lax.broadcasted_iota(jnp.int32, (SBIN, SBIN), 0)
        == lax.broadcasted_iota(jnp.int32, (SBIN, SBIN), 1), 1.0, 0.0)
    nb = jnp.sum(ii * na, axis=0, keepdims=True)
    idsr = jnp.sum(ii * ids, axis=0, keepdims=True)
    iota_l = lax.broadcasted_iota(jnp.int32, (SBIN, SBIN), 1)
    d2 = jnp.maximum(na - 2.0 * g + nb, 1e-6)
    d2 = jnp.where(iota_l < BIN, d2, 3.0e38)
    idsb = jnp.broadcast_to(idsr, (SBIN, SBIN))
    work = d2
    vals = []
    cols = []
    for _ in range(K):
        mn = jnp.min(work, axis=1, keepdims=True)
        loc = jnp.min(jnp.where(work == mn, iota_l, 10 ** 9), axis=1,
                      keepdims=True)
        sel = iota_l == loc
        colg = jnp.sum(jnp.where(sel, idsb, 0.0), axis=1, keepdims=True)
        vals.append(mn)
        cols.append(colg)
        work = jnp.where(sel, 3.4e38, work)
    d2k = jnp.concatenate(vals, axis=1)
    dmv = jnp.exp(-0.1 * jnp.sqrt(d2k))
    colk = jnp.concatenate(cols, axis=1)
    erow_ref[0] = jnp.concatenate(
        [dmv, colk, jnp.zeros((SBIN, 128 - 2 * K), jnp.float32)], axis=1)
    li = lax.broadcasted_iota(jnp.int32, (1, SBIN), 1)
    eidx_ref[0, 0] = jnp.where(
        li < BIN, idsr.astype(jnp.int32), N + GPB * i + (li - BIN)) + P * b


def _k3_call(sorted_comb):
    return pl.pallas_call(
        _k3_body,
        grid=(B, NBINS),
        in_specs=[pl.BlockSpec((1, SBIN, 128), lambda b, i: (b, i, 0))],
        out_specs=[
            pl.BlockSpec((1, SBIN, 128), lambda b, i: (b, i, 0)),
            pl.BlockSpec((1, 1, 1, SBIN), lambda b, i: (b, i, 0, 0)),
        ],
        out_shape=[
            jax.ShapeDtypeStruct((B, P, 128), jnp.float32),
            jax.ShapeDtypeStruct((B, NBINS, 1, SBIN), jnp.int32),
        ],
    )(sorted_comb)


def _e_body(ge_ref, out_ref):
    ent = ge_ref[0]
    iota_l = lax.broadcasted_iota(jnp.int32, (RB, N), 1)
    acc = jnp.zeros((RB, N), jnp.float32)
    for j in reversed(range(K)):
        v = ent[:, j:j + 1]
        c = ent[:, K + j:K + j + 1].astype(jnp.int32)
        acc = jnp.where(iota_l == c, v, acc)
    out_ref[0] = acc


def _e_call(ge):
    return pl.pallas_call(
        _e_body,
        grid=(B, N // RB),
        in_specs=[pl.BlockSpec((1, RB, 128), lambda b, r: (b, r, 0))],
        out_specs=pl.BlockSpec((1, RB, N), lambda b, r: (b, r, 0)),
        out_shape=jax.ShapeDtypeStruct((B, N, N), jnp.float32),
        compiler_params=pltpu.CompilerParams(
            dimension_semantics=("parallel", "parallel")),
    )(ge)


def kernel(points, rotations):
    rot5 = rotations[:, : NBINS // 2]
    mul = jnp.matmul(points, rot5)

    pos, comb = _k1_call(points, mul)
    t = jnp.arange(PAD, dtype=jnp.int32)
    padi = (SBIN * (t // GPB) + BIN + t % GPB)[None, :] \
        + P * jnp.arange(B, dtype=jnp.int32)[:, None]
    pflat = jnp.concatenate(
        [pos.reshape(B, N), padi], axis=1).reshape(B * P)
    sorted_comb = _sc_scatter(pflat, comb)

    erows, eidx = _k3_call(sorted_comb.reshape(B, P, 128))
    ge = _sc_scatter(eidx.reshape(B * P), erows.reshape(B * P, 128))

    return _e_call(ge.reshape(B, P, 128))
